# baseline (speedup 1.0000x reference)
import functools

import jax
import jax.numpy as jnp
from jax import lax
from jax.experimental import pallas as pl
from jax.experimental.pallas import tpu as pltpu
from jax.experimental.pallas import tpu_sc as plsc

NC, NS, L = 2, 16, 16
NW = NC * NS
B, C, H, W = 32, 3, 512, 512
PLANE = H * W
CHUNK = 8192
NCHUNK = PLANE // CHUNK
GROUPS = CHUNK // L
NBINS = 10
HWORDS = 2 * NBINS * L

_mesh = plsc.VectorSubcoreMesh(
    core_axis_name="c", subcore_axis_name="s", num_cores=NC, num_subcores=NS
)


@functools.partial(
    pl.kernel,
    out_type=jax.ShapeDtypeStruct((NW, 6, HWORDS), jnp.float32),
    mesh=_mesh,
    compiler_params=pltpu.CompilerParams(needs_layout_passes=False),
    scratch_types=[
        pltpu.VMEM((CHUNK,), jnp.float32),
        pltpu.VMEM((CHUNK,), jnp.float32),
        pltpu.VMEM((CHUNK,), jnp.float32),
    ]
    + [pltpu.VMEM((HWORDS,), jnp.float32) for _ in range(6)],
)
def _hist_kernel(real_hbm, fake_hbm, out_hbm, rbuf, gbuf, bbuf, h0, h1, s0, s1, v0, v1):
    wid = lax.axis_index("s") * NC + lax.axis_index("c")
    zero = jnp.zeros((L,), jnp.float32)
    for ref in (h0, h1, s0, s1, v0, v1):
        for i in range(HWORDS // L):
            ref[pl.ds(i * L, L)] = zero
    lanes = lax.iota(jnp.int32, L)
    ones = jnp.ones((L,), jnp.float32)
    banks = ((h0, s0, v0), (h1, s1, v1))

    def do_image(src_hbm, img_off):
        plane0 = wid * (C * PLANE)
        loff = lanes + img_off

        def chunk_body(k, carry):
            off = plane0 + k * CHUNK
            pltpu.sync_copy(src_hbm.at[pl.ds(off, CHUNK)], rbuf)
            pltpu.sync_copy(src_hbm.at[pl.ds(off + PLANE, CHUNK)], gbuf)
            pltpu.sync_copy(src_hbm.at[pl.ds(off + 2 * PLANE, CHUNK)], bbuf)

            def grp(i, c2):
                for bank, (hh, hs, hv) in enumerate(banks):
                    s = (2 * i + bank) * L
                    r = jnp.clip(rbuf[pl.ds(s, L)], 0.0, 1.0)
                    g = jnp.clip(gbuf[pl.ds(s, L)], 0.0, 1.0)
                    b = jnp.clip(bbuf[pl.ds(s, L)], 0.0, 1.0)
                    mx = jnp.maximum(r, jnp.maximum(g, b))
                    mn = jnp.minimum(r, jnp.minimum(g, b))
                    d = mx - mn
                    nz = d != 0.0
                    safe = jnp.where(nz, d, 1.0)
                    mxnz = mx != 0.0
                    safe_mx = jnp.where(mxnz, mx, 1.0)
                    invb = 1.0 / (safe * safe_mx)
                    inv = invb * safe_mx
                    inv_mx = invb * safe
                    q = (g - b) * inv
                    hr = jnp.where(q < 0.0, q + 6.0, q)
                    hg = (b - r) * inv + 2.0
                    hb = (r - g) * inv + 4.0
                    mask_r = (mx == r) & nz
                    mask_g = (mx == g) & nz
                    mask_b = (mx == b) & nz
                    hue = jnp.where(mask_b, hb, jnp.where(mask_g, hg, jnp.where(mask_r, hr, 0.0)))
                    hue = hue * (1.0 / 6.0)
                    sat = jnp.where(mxnz, d * inv_mx, 0.0)
                    bh = jnp.minimum((hue * 10.0).astype(jnp.int32), 9)
                    bs = jnp.minimum((sat * 10.0).astype(jnp.int32), 9)
                    bv = jnp.minimum((mx * 10.0).astype(jnp.int32), 9)
                    plsc.addupdate_scatter(hh, [bh * L + loff], ones)
                    plsc.addupdate_scatter(hs, [bs * L + loff], ones)
                    plsc.addupdate_scatter(hv, [bv * L + loff], ones)
                return c2

            lax.fori_loop(0, GROUPS // 2, grp, 0, unroll=2)
            return carry

        lax.fori_loop(0, NCHUNK, chunk_body, 0)

    do_image(real_hbm, 0)
    do_image(fake_hbm, NBINS * L)
    for j, ref in enumerate((h0, h1, s0, s1, v0, v1)):
        pltpu.sync_copy(ref, out_hbm.at[wid, j])


def _loss_body(hist_ref, out_ref):
    x = hist_ref[...]
    tot = jnp.sum(x, axis=1, keepdims=True)
    d = jnp.abs(tot[: 3 * NBINS] - tot[3 * NBINS :])
    w = jnp.concatenate(
        [
            jnp.full((NBINS, 1), 0.3 / NBINS, jnp.float32),
            jnp.full((NBINS, 1), 0.4 / NBINS, jnp.float32),
            jnp.full((NBINS, 1), 0.4 / NBINS, jnp.float32),
        ],
        axis=0,
    )
    out_ref[0, 0] = jnp.sum(d * w)


def kernel(x_real, x_fake):
    part = _hist_kernel(x_real.reshape(-1), x_fake.reshape(-1))
    part = part.reshape(NW, 3, 2, 2, NBINS, L)
    x = part.transpose(3, 1, 4, 0, 2, 5).reshape(6 * NBINS, NW * 2 * L)
    loss = pl.pallas_call(
        _loss_body,
        out_shape=jax.ShapeDtypeStruct((1, 1), jnp.float32),
        out_specs=pl.BlockSpec(memory_space=pltpu.SMEM),
    )(x)
    return loss[0, 0]


# speedup vs baseline: 5.8426x; 5.8426x over previous
import functools

import jax
import jax.numpy as jnp
from jax import lax
from jax.experimental import pallas as pl
from jax.experimental.pallas import tpu as pltpu
from jax.experimental.pallas import tpu_sc as plsc

NC, NS, L = 2, 16, 16
NW = NC * NS
B, C, H, W = 32, 3, 512, 512
PLANE = H * W
CROWS = 16
CHUNK = CROWS * W
NCHUNK = H // CROWS
GROUPS = CHUNK // L
GPR = W // L
NBINS = 10
HWORDS = 2 * NBINS * L
SB = 16

_mesh = plsc.VectorSubcoreMesh(
    core_axis_name="c", subcore_axis_name="s", num_cores=NC, num_subcores=NS
)


@functools.partial(
    pl.kernel,
    out_type=jax.ShapeDtypeStruct((NW, 6, HWORDS), jnp.float32),
    mesh=_mesh,
    compiler_params=pltpu.CompilerParams(needs_layout_passes=False),
    scratch_types=[pltpu.VMEM((CROWS, W), jnp.float32) for _ in range(6)]
    + [pltpu.VMEM((HWORDS,), jnp.float32) for _ in range(6)]
    + [pltpu.SemaphoreType.DMA, pltpu.SemaphoreType.DMA],
)
def _hist_kernel(
    real_hbm, fake_hbm, out_hbm, r0, g0, b0, r1, g1, b1, h0, h1, s0, s1, v0, v1, semA, semB
):
    wid = lax.axis_index("s") * NC + lax.axis_index("c")
    zero = jnp.zeros((L,), jnp.float32)
    for ref in (h0, h1, s0, s1, v0, v1):
        for i in range(HWORDS // L):
            ref[pl.ds(i * L, L)] = zero
    lanes = lax.iota(jnp.int32, L)
    ones = jnp.ones((L,), jnp.float32)
    banks = ((h0, s0, v0), (h1, s1, v1))
    bufsets = ((r0, g0, b0, semA), (r1, g1, b1, semB))

    def do_image(src_hbm, img_off, batch):
        loff = lanes + img_off

        def start_chunk(k, bufset):
            rb, gb, bb, sem = bufset
            row0 = k * CROWS
            pltpu.async_copy(src_hbm.at[batch, 0, pl.ds(row0, CROWS)], rb, sem)
            pltpu.async_copy(src_hbm.at[batch, 1, pl.ds(row0, CROWS)], gb, sem)
            pltpu.async_copy(src_hbm.at[batch, 2, pl.ds(row0, CROWS)], bb, sem)

        def compute_chunk(bufset):
            rbuf, gbuf, bbuf, sem = bufset
            for buf in (rbuf, gbuf, bbuf):
                pltpu.make_async_copy(src_hbm.at[0, 0, pl.ds(0, CROWS)], buf, sem).wait()

            @plsc.parallel_loop(0, GROUPS, step=2, unroll=4)
            def grp(i):
                for bank, (hh, hs, hv) in enumerate(banks):
                    idx = i + bank
                    row = lax.shift_right_logical(idx, 5)
                    col = lax.bitwise_and(idx, GPR - 1) * L
                    r = rbuf[row, pl.ds(col, L)]
                    g = gbuf[row, pl.ds(col, L)]
                    b = bbuf[row, pl.ds(col, L)]
                    mx = jnp.maximum(r, jnp.maximum(g, b))
                    mn = jnp.minimum(r, jnp.minimum(g, b))
                    d = mx - mn
                    nz = d != 0.0
                    safe = jnp.where(nz, d, 1.0)
                    mxnz = mx != 0.0
                    safe_mx = jnp.where(mxnz, mx, 1.0)
                    invb = 10.0 / (safe * safe_mx)
                    inv = invb * safe_mx
                    inv_mx = invb * safe
                    q = (g - b) * inv
                    hr = jnp.where(q < 0.0, q + 60.0, q)
                    hg = (b - r) * inv + 20.0
                    hb = (r - g) * inv + 40.0
                    mask_r = (mx == r) & nz
                    mask_g = (mx == g) & nz
                    mask_b = (mx == b) & nz
                    hue10 = jnp.where(mask_b, hb, jnp.where(mask_g, hg, jnp.where(mask_r, hr, 0.0)))
                    hue10 = hue10 * (1.0 / 6.0)
                    sat10 = d * inv_mx
                    bh = jnp.minimum(hue10.astype(jnp.int32), 9)
                    bs = jnp.minimum(sat10.astype(jnp.int32), 9)
                    bv = (mx * 10.0).astype(jnp.int32)
                    plsc.addupdate_scatter(hh, [bh * L + loff], ones)
                    plsc.addupdate_scatter(hs, [bs * L + loff], ones)
                    plsc.addupdate_scatter(hv, [bv * L + loff], ones)

        start_chunk(0, bufsets[0])

        def pair_body(kk, carry):
            k0 = 2 * kk
            start_chunk(k0 + 1, bufsets[1])
            compute_chunk(bufsets[0])

            @pl.when(k0 + 2 < NCHUNK)
            def _():
                start_chunk(k0 + 2, bufsets[0])

            compute_chunk(bufsets[1])
            return carry

        lax.fori_loop(0, NCHUNK // 2, pair_body, 0)

    @pl.when(wid < SB)
    def _():
        do_image(real_hbm, 0, wid)

    @pl.when(wid >= SB)
    def _():
        do_image(fake_hbm, NBINS * L, wid - SB)

    for j, ref in enumerate((h0, h1, s0, s1, v0, v1)):
        pltpu.sync_copy(ref, out_hbm.at[wid, j])


def _tc_hist_body(xr_ref, xf_ref, out_ref):
    @pl.when(pl.program_id(0) == 0)
    def _():
        for rf in range(2):
            for c in range(3):
                for k in range(NBINS):
                    out_ref[rf, c, k] = 0.0

    for rf, ref in ((0, xr_ref), (1, xf_ref)):
        r = ref[0, 0]
        g = ref[0, 1]
        b = ref[0, 2]
        mx = jnp.maximum(r, jnp.maximum(g, b))
        mn = jnp.minimum(r, jnp.minimum(g, b))
        d = mx - mn
        nz = d != 0.0
        safe = jnp.where(nz, d, 1.0)
        mxnz = mx != 0.0
        safe_mx = jnp.where(mxnz, mx, 1.0)
        invb = 10.0 / (safe * safe_mx)
        inv = invb * safe_mx
        inv_mx = invb * safe
        q = (g - b) * inv
        hr = jnp.where(q < 0.0, q + 60.0, q)
        hg = (b - r) * inv + 20.0
        hb = (r - g) * inv + 40.0
        mask_r = mx == r
        mask_g = (mx == g) & nz
        mask_b = (mx == b) & nz
        hue10 = jnp.where(mask_b, hb, jnp.where(mask_g, hg, jnp.where(mask_r, hr, 0.0)))
        hue10 = hue10 * (1.0 / 6.0)
        sat10 = d * inv_mx
        bh = jnp.minimum(hue10.astype(jnp.int32), 9)
        bs = jnp.minimum(sat10.astype(jnp.int32), 9)
        bv = (mx * 10.0).astype(jnp.int32)
        for c, bins in ((0, bh), (1, bs), (2, bv)):
            for k in range(NBINS):
                out_ref[rf, c, k] += jnp.sum((bins == k).astype(jnp.float32))


def _loss_body(hist_ref, tc_ref, out_ref):
    x = hist_ref[...]
    tot = jnp.sum(x, axis=1, keepdims=True) + tc_ref[...]
    d = jnp.abs(tot[: 3 * NBINS] - tot[3 * NBINS :])
    w = jnp.concatenate(
        [
            jnp.full((NBINS, 1), 0.3 / NBINS, jnp.float32),
            jnp.full((NBINS, 1), 0.4 / NBINS, jnp.float32),
            jnp.full((NBINS, 1), 0.4 / NBINS, jnp.float32),
        ],
        axis=0,
    )
    out_ref[0, 0] = jnp.sum(d * w)


def kernel(x_real, x_fake):
    part = _hist_kernel(x_real, x_fake)
    tc_part = pl.pallas_call(
        _tc_hist_body,
        grid=(B - SB,),
        in_specs=[
            pl.BlockSpec((1, C, H, W), lambda i: (i + SB, 0, 0, 0)),
            pl.BlockSpec((1, C, H, W), lambda i: (i + SB, 0, 0, 0)),
        ],
        out_specs=pl.BlockSpec((2, 3, NBINS), lambda i: (0, 0, 0), memory_space=pltpu.SMEM),
        out_shape=jax.ShapeDtypeStruct((2, 3, NBINS), jnp.float32),
    )(x_real, x_fake)
    part = part.reshape(NW, 3, 2, 2, NBINS, L)
    x = part.transpose(3, 1, 4, 0, 2, 5).reshape(6 * NBINS, NW * 2 * L)
    y = tc_part.reshape(6 * NBINS, 1)
    loss = pl.pallas_call(
        _loss_body,
        out_shape=jax.ShapeDtypeStruct((1, 1), jnp.float32),
        out_specs=pl.BlockSpec(memory_space=pltpu.SMEM),
    )(x, y)
    return loss[0, 0]
